# baseline (speedup 1.0000x reference)
import functools

import jax
import jax.numpy as jnp
from jax import lax
from jax.experimental import pallas as pl
from jax.experimental.pallas import tpu as pltpu
from jax.experimental.pallas import tpu_sc as plsc

B = 4096
S = 200
D = 64
DP = 128
HIDDEN = 300

NC = 2
NS = 16
NW = NC * NS
SAMP_PER_W = B // NW
CHUNK = S // 2
CHUNKS_PER_W = SAMP_PER_W * 2
NBUF = 4

_sc_mesh = plsc.VectorSubcoreMesh(
    core_axis_name="c", subcore_axis_name="s", num_cores=NC, num_subcores=NS
)


def _pool_body(x_hbm, table_hbm, out_hbm, idx_v, rows_v, out_v, *sems):
    w = lax.axis_index("s") * NC + lax.axis_index("c")

    pltpu.sync_copy(x_hbm.at[w], idx_v)

    def gather(g, b):
        pltpu.async_copy(
            table_hbm.at[idx_v.at[g, pl.ds(0, CHUNK)]], rows_v.at[b], sems[b]
        )

    for b in range(NBUF):
        gather(b, b)

    def outer(it, carry):
        for half in range(NBUF // 2):
            smp = it * (NBUF // 2) + half
            accs = tuple(jnp.zeros((16,), jnp.float32) for _ in range(4))
            for p in range(2):
                b = half * 2 + p
                pltpu.make_async_copy(
                    table_hbm.at[idx_v.at[0, pl.ds(0, CHUNK)]],
                    rows_v.at[b],
                    sems[b],
                ).wait()

                def row_body(r, a, b=b):
                    return tuple(
                        a[k] + rows_v[b, r, pl.ds(16 * k, 16)] for k in range(4)
                    )

                accs = lax.fori_loop(0, CHUNK, row_body, accs, unroll=4)

                g_next = (it + 1) * NBUF + b

                @pl.when(g_next < CHUNKS_PER_W)
                def _(b=b, g_next=g_next):
                    gather(g_next, b)

            inv = jnp.float32(1.0 / S)
            for k in range(4):
                out_v[smp, pl.ds(16 * k, 16)] = accs[k] * inv
        return carry

    lax.fori_loop(0, CHUNKS_PER_W // NBUF, outer, 0)

    pltpu.sync_copy(out_v, out_hbm.at[pl.ds(w * SAMP_PER_W, SAMP_PER_W)])


_sc_pool = pl.kernel(
    _pool_body,
    out_type=jax.ShapeDtypeStruct((B, DP), jnp.float32),
    mesh=_sc_mesh,
    scratch_types=[
        pltpu.VMEM((CHUNKS_PER_W, DP), jnp.int32),
        pltpu.VMEM((NBUF, CHUNK, DP), jnp.float32),
        pltpu.VMEM((SAMP_PER_W, DP), jnp.float32),
    ]
    + [pltpu.SemaphoreType.DMA] * NBUF,
)


def _mlp_body(x_ref, w1_ref, b1_ref, w2_ref, b2_ref, w3_ref, b3_ref, o_ref):
    x = x_ref[...]
    h = jnp.maximum(
        lax.dot_general(
            x, w1_ref[...], (((1,), (0,)), ((), ())),
            preferred_element_type=jnp.float32,
        )
        + b1_ref[...],
        0.0,
    )
    h = jnp.maximum(
        lax.dot_general(
            h, w2_ref[...], (((1,), (0,)), ((), ())),
            preferred_element_type=jnp.float32,
        )
        + b2_ref[...],
        0.0,
    )
    logits = (
        lax.dot_general(
            h, w3_ref[...], (((1,), (0,)), ((), ())),
            preferred_element_type=jnp.float32,
        )
        + b3_ref[...]
    )
    m = jnp.max(logits, axis=1, keepdims=True)
    lse = m + jnp.log(jnp.sum(jnp.exp(logits - m), axis=1, keepdims=True))
    o_ref[...] = logits - lse


_MLP_BB = 512


def _mlp(pooled, W1, b1, W2, b2, W3, b3):
    grid = (B // _MLP_BB,)
    return pl.pallas_call(
        _mlp_body,
        grid=grid,
        in_specs=[
            pl.BlockSpec((_MLP_BB, D), lambda i: (i, 0)),
            pl.BlockSpec((D, HIDDEN), lambda i: (0, 0)),
            pl.BlockSpec((1, HIDDEN), lambda i: (0, 0)),
            pl.BlockSpec((HIDDEN, HIDDEN), lambda i: (0, 0)),
            pl.BlockSpec((1, HIDDEN), lambda i: (0, 0)),
            pl.BlockSpec((HIDDEN, 2), lambda i: (0, 0)),
            pl.BlockSpec((1, 2), lambda i: (0, 0)),
        ],
        out_specs=pl.BlockSpec((_MLP_BB, 2), lambda i: (i, 0)),
        out_shape=jax.ShapeDtypeStruct((B, 2), jnp.float32),
    )(pooled, W1, b1, W2, b2, W3, b3)


def kernel(x, table, W1, b1, W2, b2, W3, b3):
    table_p = jnp.pad(table, ((0, 0), (0, DP - D)))
    xr = jnp.pad(
        x.reshape(NW, CHUNKS_PER_W, CHUNK), ((0, 0), (0, 0), (0, DP - CHUNK))
    )
    pooled = _sc_pool(xr, table_p)[:, :D]
    return _mlp(
        pooled, W1, b1.reshape(1, HIDDEN), W2, b2.reshape(1, HIDDEN),
        W3, b3.reshape(1, 2),
    )


# speedup vs baseline: 1.0199x; 1.0199x over previous
import jax
import jax.numpy as jnp
from jax import lax
from jax.experimental import pallas as pl
from jax.experimental.pallas import tpu as pltpu
from jax.experimental.pallas import tpu_sc as plsc

B = 4096
S = 200
D = 64
HIDDEN = 300
VOCAB_ROWS = 1000000

NC = 2
NS = 16
NW = NC * NS
SAMP_PER_W = B // NW
CHUNK = S // 2
CHUNKS_PER_W = SAMP_PER_W * 2
NBUF = 4

_sc_mesh = plsc.VectorSubcoreMesh(
    core_axis_name="c", subcore_axis_name="s", num_cores=NC, num_subcores=NS
)


_CBLK = 4000
_HALF = VOCAB_ROWS // 2


def _compact_body(a_ref, b_ref, o_ref):
    o_ref[:, 0:D] = a_ref[...]
    o_ref[:, D : 2 * D] = b_ref[...]


def _compact(table):
    nblk = _HALF // _CBLK
    return pl.pallas_call(
        _compact_body,
        grid=(nblk,),
        in_specs=[
            pl.BlockSpec((_CBLK, D), lambda i: (i, 0)),
            pl.BlockSpec((_CBLK, D), lambda i, nblk=nblk: (i + nblk, 0)),
        ],
        out_specs=pl.BlockSpec((_CBLK, 2 * D), lambda i: (i, 0)),
        out_shape=jax.ShapeDtypeStruct((_HALF, 2 * D), jnp.float32),
    )(table, table)


def _pool_body(x_hbm, table_hbm, out_hbm, idx_v, rows_v, out_v, *sems):
    w = lax.axis_index("s") * NC + lax.axis_index("c")

    pltpu.sync_copy(x_hbm.at[w], idx_v)

    for b in range(NBUF):
        pltpu.async_copy(table_hbm.at[idx_v.at[b]], rows_v.at[b], sems[b])

    def outer(it, carry):
        for half in range(NBUF // 2):
            smp = it * (NBUF // 2) + half
            accs = tuple(jnp.zeros((16,), jnp.float32) for _ in range(4))
            for p in range(2):
                b = half * 2 + p
                pltpu.make_async_copy(
                    table_hbm.at[idx_v.at[b]], rows_v.at[b], sems[b]
                ).wait()

                def row_body(r, a, b=b):
                    return tuple(
                        a[k] + rows_v[b, r, pl.ds(16 * k, 16)] for k in range(4)
                    )

                accs = lax.fori_loop(0, CHUNK, row_body, accs, unroll=4)

                g_next = (it + 1) * NBUF + b

                @pl.when(g_next < CHUNKS_PER_W)
                def _(b=b, g_next=g_next):
                    pltpu.async_copy(
                        table_hbm.at[idx_v.at[g_next]], rows_v.at[b], sems[b]
                    )

            inv = jnp.float32(1.0 / S)
            for k in range(4):
                out_v[smp, pl.ds(16 * k, 16)] = accs[k] * inv
        return carry

    lax.fori_loop(0, CHUNKS_PER_W // NBUF, outer, 0)

    pltpu.sync_copy(out_v, out_hbm.at[pl.ds(w * SAMP_PER_W, SAMP_PER_W)])


_sc_pool = pl.kernel(
    _pool_body,
    out_type=jax.ShapeDtypeStruct((B, D), jnp.float32),
    mesh=_sc_mesh,
    scratch_types=[
        pltpu.VMEM((CHUNKS_PER_W, CHUNK), jnp.int32),
        pltpu.VMEM((NBUF, CHUNK, D), jnp.float32),
        pltpu.VMEM((SAMP_PER_W, D), jnp.float32),
    ]
    + [pltpu.SemaphoreType.DMA] * NBUF,
    compiler_params=pltpu.CompilerParams(use_tc_tiling_on_sc=False),
)


def _mlp_body(x_ref, w1_ref, b1_ref, w2_ref, b2_ref, w3_ref, b3_ref, o_ref):
    x = x_ref[...]
    h = jnp.maximum(
        lax.dot_general(
            x, w1_ref[...], (((1,), (0,)), ((), ())),
            preferred_element_type=jnp.float32,
        )
        + b1_ref[...],
        0.0,
    )
    h = jnp.maximum(
        lax.dot_general(
            h, w2_ref[...], (((1,), (0,)), ((), ())),
            preferred_element_type=jnp.float32,
        )
        + b2_ref[...],
        0.0,
    )
    logits = (
        lax.dot_general(
            h, w3_ref[...], (((1,), (0,)), ((), ())),
            preferred_element_type=jnp.float32,
        )
        + b3_ref[...]
    )
    m = jnp.max(logits, axis=1, keepdims=True)
    lse = m + jnp.log(jnp.sum(jnp.exp(logits - m), axis=1, keepdims=True))
    o_ref[...] = logits - lse


_MLP_BB = 512


def _mlp(pooled, W1, b1, W2, b2, W3, b3):
    grid = (B // _MLP_BB,)
    return pl.pallas_call(
        _mlp_body,
        grid=grid,
        in_specs=[
            pl.BlockSpec((_MLP_BB, D), lambda i: (i, 0)),
            pl.BlockSpec((D, HIDDEN), lambda i: (0, 0)),
            pl.BlockSpec((1, HIDDEN), lambda i: (0, 0)),
            pl.BlockSpec((HIDDEN, HIDDEN), lambda i: (0, 0)),
            pl.BlockSpec((1, HIDDEN), lambda i: (0, 0)),
            pl.BlockSpec((HIDDEN, 2), lambda i: (0, 0)),
            pl.BlockSpec((1, 2), lambda i: (0, 0)),
        ],
        out_specs=pl.BlockSpec((_MLP_BB, 2), lambda i: (i, 0)),
        out_shape=jax.ShapeDtypeStruct((B, 2), jnp.float32),
    )(pooled, W1, b1, W2, b2, W3, b3)


def kernel(x, table, W1, b1, W2, b2, W3, b3):
    table_lin = _compact(table).reshape(VOCAB_ROWS, D)
    xf = jnp.where(x < _HALF, x * 2, (x - _HALF) * 2 + 1)
    xr = xf.reshape(NW, CHUNKS_PER_W, CHUNK)
    pooled = _sc_pool(xr, table_lin)
    return _mlp(
        pooled, W1, b1.reshape(1, HIDDEN), W2, b2.reshape(1, HIDDEN),
        W3, b3.reshape(1, 2),
    )


# speedup vs baseline: 1.0258x; 1.0058x over previous
import jax
import jax.numpy as jnp
from jax import lax
from jax.experimental import pallas as pl
from jax.experimental.pallas import tpu as pltpu
from jax.experimental.pallas import tpu_sc as plsc

B = 4096
S = 200
D = 64
HIDDEN = 300
VOCAB_ROWS = 1000000

NC = 2
NS = 16
NW = NC * NS
SAMP_PER_W = B // NW
CHUNK = S // 2
CHUNKS_PER_W = SAMP_PER_W * 2
NBUF = 4

_sc_mesh = plsc.VectorSubcoreMesh(
    core_axis_name="c", subcore_axis_name="s", num_cores=NC, num_subcores=NS
)


_CBLK = 4000
_HALF = VOCAB_ROWS // 2


def _compact_body(a_ref, b_ref, o_ref):
    o_ref[:, 0:D] = a_ref[...]
    o_ref[:, D : 2 * D] = b_ref[...]


def _compact(table):
    nblk = _HALF // _CBLK
    return pl.pallas_call(
        _compact_body,
        grid=(nblk,),
        in_specs=[
            pl.BlockSpec((_CBLK, D), lambda i: (i, 0)),
            pl.BlockSpec((_CBLK, D), lambda i, nblk=nblk: (i + nblk, 0)),
        ],
        out_specs=pl.BlockSpec((_CBLK, 2 * D), lambda i: (i, 0)),
        out_shape=jax.ShapeDtypeStruct((_HALF, 2 * D), jnp.float32),
    )(table, table)


def _pool_body(x_hbm, table_hbm, out_hbm, idx_v, rows_v, out_v, *sems):
    w = lax.axis_index("s") * NC + lax.axis_index("c")

    pltpu.sync_copy(x_hbm.at[w], idx_v)

    for b in range(NBUF):
        pltpu.async_copy(table_hbm.at[idx_v.at[b]], rows_v.at[b], sems[b])

    def outer(it, carry):
        for half in range(NBUF // 2):
            smp = it * (NBUF // 2) + half
            accs = tuple(jnp.zeros((16,), jnp.float32) for _ in range(4))
            for p in range(2):
                b = half * 2 + p
                pltpu.make_async_copy(
                    table_hbm.at[idx_v.at[b]], rows_v.at[b], sems[b]
                ).wait()

                def row_body(r, a, b=b):
                    return tuple(
                        a[k] + rows_v[b, r, pl.ds(16 * k, 16)] for k in range(4)
                    )

                accs = lax.fori_loop(0, CHUNK, row_body, accs, unroll=4)

                g_next = (it + 1) * NBUF + b

                @pl.when(g_next < CHUNKS_PER_W)
                def _(b=b, g_next=g_next):
                    pltpu.async_copy(
                        table_hbm.at[idx_v.at[g_next]], rows_v.at[b], sems[b]
                    )

            inv = jnp.float32(1.0 / S)
            for k in range(4):
                out_v[smp, pl.ds(16 * k, 16)] = accs[k] * inv
        return carry

    lax.fori_loop(0, CHUNKS_PER_W // NBUF, outer, 0)

    pltpu.sync_copy(out_v, out_hbm.at[pl.ds(w * SAMP_PER_W, SAMP_PER_W)])


_sc_pool = pl.kernel(
    _pool_body,
    out_type=jax.ShapeDtypeStruct((B, D), jnp.float32),
    mesh=_sc_mesh,
    scratch_types=[
        pltpu.VMEM((CHUNKS_PER_W, CHUNK), jnp.int32),
        pltpu.VMEM((NBUF, CHUNK, D), jnp.float32),
        pltpu.VMEM((SAMP_PER_W, D), jnp.float32),
    ]
    + [pltpu.SemaphoreType.DMA] * NBUF,
    compiler_params=pltpu.CompilerParams(use_tc_tiling_on_sc=False),
)


def _mlp_body(x_ref, w1_ref, b1_ref, w2_ref, b2_ref, w3_ref, b3_ref, o_ref):
    x = x_ref[...]
    h = jnp.maximum(
        lax.dot_general(
            x, w1_ref[...], (((1,), (0,)), ((), ())),
            preferred_element_type=jnp.float32,
        )
        + b1_ref[...],
        0.0,
    )
    h = jnp.maximum(
        lax.dot_general(
            h, w2_ref[...], (((1,), (0,)), ((), ())),
            preferred_element_type=jnp.float32,
        )
        + b2_ref[...],
        0.0,
    )
    logits = (
        lax.dot_general(
            h, w3_ref[...], (((1,), (0,)), ((), ())),
            preferred_element_type=jnp.float32,
        )
        + b3_ref[...]
    )
    m = jnp.max(logits, axis=1, keepdims=True)
    lse = m + jnp.log(jnp.sum(jnp.exp(logits - m), axis=1, keepdims=True))
    o_ref[...] = logits - lse


_MLP_BB = 512


def _mlp(pooled, W1, b1, W2, b2, W3, b3):
    grid = (B // _MLP_BB,)
    return pl.pallas_call(
        _mlp_body,
        grid=grid,
        in_specs=[
            pl.BlockSpec((_MLP_BB, D), lambda i: (i, 0)),
            pl.BlockSpec((D, HIDDEN), lambda i: (0, 0)),
            pl.BlockSpec((1, HIDDEN), lambda i: (0, 0)),
            pl.BlockSpec((HIDDEN, HIDDEN), lambda i: (0, 0)),
            pl.BlockSpec((1, HIDDEN), lambda i: (0, 0)),
            pl.BlockSpec((HIDDEN, 2), lambda i: (0, 0)),
            pl.BlockSpec((1, 2), lambda i: (0, 0)),
        ],
        out_specs=pl.BlockSpec((_MLP_BB, 2), lambda i: (i, 0)),
        out_shape=jax.ShapeDtypeStruct((B, 2), jnp.float32),
    )(pooled, W1, b1, W2, b2, W3, b3)


def kernel(x, table, W1, b1, W2, b2, W3, b3):
    table_lin = table + jnp.zeros((1, 1), jnp.float32)
    xr = x.reshape(NW, CHUNKS_PER_W, CHUNK)
    pooled = _sc_pool(xr, table_lin)
    return _mlp(
        pooled, W1, b1.reshape(1, HIDDEN), W2, b2.reshape(1, HIDDEN),
        W3, b3.reshape(1, 2),
    )
